# Initial kernel scaffold; baseline (speedup 1.0000x reference)
import jax
import jax.numpy as jnp
from jax.experimental import pallas as pl


def kernel(inputs):
    raise NotImplementedError("write your pallas kernel here")


# speedup vs baseline: 18.3790x; 18.3790x over previous
import functools

import jax
import jax.numpy as jnp
from jax import lax
from jax.experimental import pallas as pl
from jax.experimental.pallas import tpu as pltpu
from jax.experimental.pallas import tpu_sc as plsc

K = 8
B, S, C = 4, 8192, 768
L = 16
NC, NS = 2, 16
NW = NC * NS
CG = C // L
TASKS = B * CG
TPW = TASKS // NW
CHUNK = 2048
NCHUNK = S // CHUNK
G = 16
NGROUP = CHUNK // G


def _insert(state, v):
    out = []
    for s in state:
        hi = jnp.maximum(s, v)
        v = jnp.minimum(s, v)
        out.append(hi)
    return tuple(out)


@functools.partial(
    pl.kernel,
    mesh=plsc.VectorSubcoreMesh(core_axis_name="c", subcore_axis_name="s"),
    out_type=jax.ShapeDtypeStruct((B, C * K), jnp.float32),
    scratch_types=[
        pltpu.VMEM((CHUNK, L), jnp.float32),
        pltpu.VMEM((L * K,), jnp.float32),
    ],
    compiler_params=pltpu.CompilerParams(
        use_tc_tiling_on_sc=False, needs_layout_passes=False),
)
def _kmax_kernel(x_hbm, out_hbm, buf, obuf):
    wid = lax.axis_index("s") * NC + lax.axis_index("c")

    def task_body(it, _carry):
        t = wid * TPW + it
        b = t // CG
        cg = t % CG

        neg = jnp.full((L,), -jnp.inf, dtype=jnp.float32)
        state0 = (neg,) * K

        def chunk_body(ci, state):
            pltpu.sync_copy(
                x_hbm.at[b, pl.ds(ci * CHUNK, CHUNK), pl.ds(cg * L, L)],
                buf)

            def group_body(g, st):
                base = g * G
                rows = [buf[base + r] for r in range(G)]
                m = list(rows)
                while len(m) > 1:
                    m = [jnp.maximum(m[2 * i], m[2 * i + 1])
                         for i in range(len(m) // 2)]
                pred = jnp.any(m[0] > st[K - 1])

                def do(ops):
                    st2, rws = ops[:K], ops[K:]
                    for r in rws:
                        st2 = _insert(st2, r)
                    return st2

                def dont(ops):
                    return ops[:K]

                return lax.cond(pred, do, dont, tuple(st) + tuple(rows))

            return lax.fori_loop(0, NGROUP, group_body, state)

        state = lax.fori_loop(0, NCHUNK, chunk_body, state0)

        lanes = lax.broadcasted_iota(jnp.int32, (L,), 0)
        for j in range(K):
            plsc.store_scatter(obuf, [lanes * K + j], state[j])
        pltpu.sync_copy(obuf, out_hbm.at[b, pl.ds(cg * (L * K), L * K)])
        return _carry

    lax.fori_loop(0, TPW, task_body, 0)


def kernel(inputs):
    return _kmax_kernel(inputs)


# speedup vs baseline: 22.5694x; 1.2280x over previous
import functools

import jax
import jax.numpy as jnp
from jax import lax
from jax.experimental import pallas as pl
from jax.experimental.pallas import tpu as pltpu
from jax.experimental.pallas import tpu_sc as plsc

K = 8
B, S, C = 4, 8192, 768
L = 16
NC, NS = 2, 16
NW = NC * NS
CG = C // L
TASKS = B * CG
TPW = TASKS // NW
CHUNK = 2048
NCHUNK = S // CHUNK
G = 16
NGROUP = CHUNK // G


def _insert(state, v):
    out = []
    for s in state:
        hi = jnp.maximum(s, v)
        v = jnp.minimum(s, v)
        out.append(hi)
    return tuple(out)


@functools.partial(
    pl.kernel,
    mesh=plsc.VectorSubcoreMesh(core_axis_name="c", subcore_axis_name="s"),
    out_type=jax.ShapeDtypeStruct((B, C * K), jnp.float32),
    scratch_types=[
        pltpu.VMEM((CHUNK, L), jnp.float32),
        pltpu.VMEM((CHUNK, L), jnp.float32),
        pltpu.VMEM((L * K,), jnp.float32),
        pltpu.SemaphoreType.DMA,
        pltpu.SemaphoreType.DMA,
    ],
    compiler_params=pltpu.CompilerParams(
        use_tc_tiling_on_sc=False, needs_layout_passes=False),
)
def _kmax_kernel(x_hbm, out_hbm, buf0, buf1, obuf, sem0, sem1):
    wid = lax.axis_index("s") * NC + lax.axis_index("c")

    def src(t, ci):
        b = t // CG
        cg = t % CG
        return x_hbm.at[b, pl.ds(ci * CHUNK, CHUNK), pl.ds(cg * L, L)]

    def make_group_body(buf):
        def group_body(g, st):
            base = g * G
            rows = [buf[base + r] for r in range(G)]
            m = list(rows)
            while len(m) > 1:
                m = [jnp.maximum(m[2 * i], m[2 * i + 1])
                     for i in range(len(m) // 2)]
            pred = jnp.any(m[0] > st[K - 1])

            def do(ops):
                st2, rws = ops[:K], ops[K:]
                for r in rws:
                    st2 = _insert(st2, r)
                return st2

            def dont(ops):
                return ops[:K]

            return lax.cond(pred, do, dont, tuple(st) + tuple(rows))
        return group_body

    bufs = ((buf0, sem0), (buf1, sem1))
    pltpu.async_copy(src(wid * TPW, 0), buf0, sem0)

    def task_body(it, _carry):
        t = wid * TPW + it
        b = t // CG
        cg = t % CG

        neg = jnp.full((L,), -jnp.inf, dtype=jnp.float32)
        state = (neg,) * K

        for ci in range(NCHUNK):
            buf, sem = bufs[ci % 2]
            nbuf, nsem = bufs[(ci + 1) % 2]
            pltpu.make_async_copy(src(t, ci), buf, sem).wait()
            if ci < NCHUNK - 1:
                pltpu.async_copy(src(t, ci + 1), nbuf, nsem)
            else:
                @pl.when(it < TPW - 1)
                def _():
                    pltpu.async_copy(src(t + 1, 0), nbuf, nsem)
            state = lax.fori_loop(0, NGROUP, make_group_body(buf), state)

        lanes = lax.broadcasted_iota(jnp.int32, (L,), 0)
        for j in range(K):
            plsc.store_scatter(obuf, [lanes * K + j], state[j])
        pltpu.sync_copy(obuf, out_hbm.at[b, pl.ds(cg * (L * K), L * K)])
        return _carry

    lax.fori_loop(0, TPW, task_body, 0)


def kernel(inputs):
    return _kmax_kernel(inputs)


# speedup vs baseline: 31.5151x; 1.3964x over previous
import functools

import jax
import jax.numpy as jnp
from jax import lax
from jax.experimental import pallas as pl
from jax.experimental.pallas import tpu as pltpu
from jax.experimental.pallas import tpu_sc as plsc

K = 8
B, S, C = 4, 8192, 768
L = 16
NC, NS = 2, 16
NW = NC * NS
CG = C // L
TASKS = B * CG
TPW = TASKS // NW
CHUNK = 2048
NCHUNK = S // CHUNK
G = 16
NGROUP = CHUNK // G


def _insert(state, v):
    out = []
    for s in state:
        hi = jnp.maximum(s, v)
        v = jnp.minimum(s, v)
        out.append(hi)
    return tuple(out)


def _merge(sa, sb):
    for r in sb:
        sa = _insert(sa, r)
    return sa


def _tree_max(rows):
    m = list(rows)
    while len(m) > 1:
        m = [jnp.maximum(m[2 * i], m[2 * i + 1]) for i in range(len(m) // 2)]
    return m[0]


@functools.partial(
    pl.kernel,
    mesh=plsc.VectorSubcoreMesh(core_axis_name="c", subcore_axis_name="s"),
    out_type=jax.ShapeDtypeStruct((B, C * K), jnp.float32),
    scratch_types=[
        pltpu.VMEM((CHUNK, L), jnp.float32),
        pltpu.VMEM((CHUNK, L), jnp.float32),
        pltpu.VMEM((NGROUP, L), jnp.float32),
        pltpu.VMEM((K, L), jnp.int32),
        pltpu.VMEM((L * K,), jnp.float32),
        pltpu.SemaphoreType.DMA,
        pltpu.SemaphoreType.DMA,
    ],
    compiler_params=pltpu.CompilerParams(
        use_tc_tiling_on_sc=False, needs_layout_passes=False),
)
def _kmax_kernel(x_hbm, out_hbm, buf0, buf1, gbuf, idxbuf, obuf, sem0, sem1):
    wid = lax.axis_index("s") * NC + lax.axis_index("c")
    lanes = lax.broadcasted_iota(jnp.int32, (L,), 0)
    neg = jnp.full((L,), -jnp.inf, dtype=jnp.float32)

    def src(t, ci):
        b = t // CG
        cg = t % CG
        return x_hbm.at[b, pl.ds(ci * CHUNK, CHUNK), pl.ds(cg * L, L)]

    def process_chunk(buf, sa, sb):
        def p1(g, thr):
            ta, tb = thr[:K], thr[K:]
            base = g * (2 * G)
            g0 = _tree_max([buf[base + r] for r in range(G)])
            g1 = _tree_max([buf[base + G + r] for r in range(G)])
            gbuf[2 * g] = g0
            gbuf[2 * g + 1] = g1
            return _insert(ta, g0) + _insert(tb, g1)

        thr = lax.fori_loop(0, NGROUP // 2, p1, ((neg,) * K) * 2)
        t8 = _merge(thr[:K], thr[K:])[K - 1]

        def p2_strict(g, cnt):
            m = (gbuf[g] > t8) & (cnt < K)
            plsc.store_scatter(idxbuf, [cnt, lanes],
                               jnp.full((L,), g, dtype=jnp.int32), mask=m)
            return cnt + m.astype(jnp.int32)

        cnt = lax.fori_loop(0, NGROUP, p2_strict, jnp.zeros((L,), jnp.int32))

        def p2_ties(g, cnt):
            m = (gbuf[g] == t8) & (cnt < K)
            plsc.store_scatter(idxbuf, [cnt, lanes],
                               jnp.full((L,), g, dtype=jnp.int32), mask=m)
            return cnt + m.astype(jnp.int32)

        lax.fori_loop(0, NGROUP, p2_ties, cnt)

        def p3(j, st):
            a, b2 = st[:K], st[K:]
            rb = idxbuf[j] * G
            for r in range(G // 2):
                a = _insert(a, plsc.load_gather(buf, [rb + r, lanes]))
            for r in range(G // 2, G):
                b2 = _insert(b2, plsc.load_gather(buf, [rb + r, lanes]))
            return a + b2

        st = lax.fori_loop(0, K, p3, tuple(sa) + tuple(sb))
        return st[:K], st[K:]

    bufs = ((buf0, sem0), (buf1, sem1))
    pltpu.async_copy(src(wid * TPW, 0), buf0, sem0)

    def task_body(it, _carry):
        t = wid * TPW + it
        b = t // CG
        cg = t % CG
        sa = (neg,) * K
        sb = (neg,) * K

        for ci in range(NCHUNK):
            buf, sem = bufs[ci % 2]
            nbuf, nsem = bufs[(ci + 1) % 2]
            pltpu.make_async_copy(src(t, ci), buf, sem).wait()
            if ci < NCHUNK - 1:
                pltpu.async_copy(src(t, ci + 1), nbuf, nsem)
            else:
                @pl.when(it < TPW - 1)
                def _():
                    pltpu.async_copy(src(t + 1, 0), nbuf, nsem)
            sa, sb = process_chunk(buf, sa, sb)

        state = _merge(sa, sb)
        for j in range(K):
            plsc.store_scatter(obuf, [lanes * K + j], state[j])
        pltpu.sync_copy(obuf, out_hbm.at[b, pl.ds(cg * (L * K), L * K)])
        return _carry

    lax.fori_loop(0, TPW, task_body, 0)


def kernel(inputs):
    return _kmax_kernel(inputs)
